# baseline (speedup 1.0000x reference)
import functools

import jax
import jax.numpy as jnp
from jax import lax
from jax.experimental import pallas as pl
from jax.experimental.pallas import tpu as pltpu
from jax.experimental.pallas import tpu_sc as plsc


def _make_emb(N, D, NC, NS):
    NW = NC * NS
    n_per_w = N // NW
    CH = 8
    n_ch = n_per_w // CH
    mesh = plsc.VectorSubcoreMesh(core_axis_name="c", subcore_axis_name="s")

    K = 7

    @functools.partial(
        pl.kernel,
        mesh=mesh,
        out_type=jax.ShapeDtypeStruct((N, D), jnp.float32),
        scratch_types=[
            pltpu.VMEM((n_per_w,), jnp.int32),
            pltpu.VMEM((K, CH, D), jnp.float32),
            pltpu.SemaphoreType.DMA((K,)),
            pltpu.SemaphoreType.DMA((K,)),
        ],
    )
    def emb(idx_hbm, table_hbm, out_hbm, idx_v, rows_v, gsem, ssem):
        wid = lax.axis_index("s") * NC + lax.axis_index("c")
        base = wid * n_per_w
        pltpu.sync_copy(idx_hbm.at[pl.ds(base, n_per_w)], idx_v)

        def gstart(c):
            b = c % K
            return pltpu.async_copy(
                table_hbm.at[idx_v.at[pl.ds(c * CH, CH)]],
                rows_v.at[b], gsem.at[b])

        def sstart(c):
            b = c % K
            return pltpu.async_copy(
                rows_v.at[b], out_hbm.at[pl.ds(base + c * CH, CH)],
                ssem.at[b])

        g = {}
        s = {}
        for c in range(min(K - 1, n_ch)):
            g[c] = gstart(c)
        for c in range(n_ch):
            g[c].wait()
            s[c] = sstart(c)
            m = c + K - 1
            if m < n_ch:
                if c >= 1:
                    s[c - 1].wait()
                g[m] = gstart(m)
        for c in range(max(0, n_ch - K), n_ch):
            s[c].wait()

    return emb


def kernel(x, weights):
    B, S = x.shape
    V, D = weights.shape
    N = B * S
    info = plsc.get_sparse_core_info()
    emb = _make_emb(N, D, info.num_cores, info.num_subcores)
    out = emb(x.reshape(N), weights)
    return out.reshape(B, 1, S, D)


# speedup vs baseline: 1.0064x; 1.0064x over previous
import functools

import jax
import jax.numpy as jnp
from jax import lax
from jax.experimental import pallas as pl
from jax.experimental.pallas import tpu as pltpu
from jax.experimental.pallas import tpu_sc as plsc


def _make_emb(N, D, NC, NS):
    NW = NC * NS
    n_per_w = N // NW
    CH = 8
    n_ch = n_per_w // CH
    mesh = plsc.VectorSubcoreMesh(core_axis_name="c", subcore_axis_name="s")

    K = 6

    @functools.partial(
        pl.kernel,
        mesh=mesh,
        out_type=jax.ShapeDtypeStruct((N, D), jnp.float32),
        scratch_types=[
            pltpu.VMEM((n_per_w,), jnp.int32),
            pltpu.VMEM((K, CH, D), jnp.float32),
            pltpu.SemaphoreType.DMA((K,)),
            pltpu.SemaphoreType.DMA((K,)),
        ],
    )
    def emb(idx_hbm, table_hbm, out_hbm, idx_v, rows_v, gsem, ssem):
        wid = lax.axis_index("s") * NC + lax.axis_index("c")
        base = wid * n_per_w
        pltpu.sync_copy(idx_hbm.at[pl.ds(base, n_per_w)], idx_v)

        def gstart(c):
            b = c % K
            return pltpu.async_copy(
                table_hbm.at[idx_v.at[pl.ds(c * CH, CH)]],
                rows_v.at[b], gsem.at[b])

        def sstart(c):
            b = c % K
            return pltpu.async_copy(
                rows_v.at[b], out_hbm.at[pl.ds(base + c * CH, CH)],
                ssem.at[b])

        g = {}
        s = {}
        for c in range(min(K - 1, n_ch)):
            g[c] = gstart(c)
        for c in range(n_ch):
            g[c].wait()
            s[c] = sstart(c)
            m = c + K - 1
            if m < n_ch:
                if c >= 1:
                    s[c - 1].wait()
                g[m] = gstart(m)
        for c in range(max(0, n_ch - K), n_ch):
            s[c].wait()

    return emb


def kernel(x, weights):
    B, S = x.shape
    V, D = weights.shape
    N = B * S
    info = plsc.get_sparse_core_info()
    emb = _make_emb(N, D, info.num_cores, info.num_subcores)
    out = emb(x.reshape(N), weights)
    return out.reshape(B, 1, S, D)


# speedup vs baseline: 1.0085x; 1.0021x over previous
import functools

import jax
import jax.numpy as jnp
from jax import lax
from jax.experimental import pallas as pl
from jax.experimental.pallas import tpu as pltpu
from jax.experimental.pallas import tpu_sc as plsc


def _make_emb(B, S, D, NC, NS):
    N = B * S
    NW = NC * NS
    n_per_w = N // NW
    w_per_row = S // n_per_w
    CH = 8
    n_ch = n_per_w // CH
    mesh = plsc.VectorSubcoreMesh(core_axis_name="c", subcore_axis_name="s")

    K = 6

    @functools.partial(
        pl.kernel,
        mesh=mesh,
        out_type=jax.ShapeDtypeStruct((N, D), jnp.float32),
        scratch_types=[
            pltpu.VMEM((n_per_w,), jnp.int32),
            pltpu.VMEM((K, CH, D), jnp.float32),
            pltpu.SemaphoreType.DMA((K,)),
            pltpu.SemaphoreType.DMA((K,)),
        ],
    )
    def emb(idx_hbm, table_hbm, out_hbm, idx_v, rows_v, gsem, ssem):
        wid = lax.axis_index("s") * NC + lax.axis_index("c")
        base = wid * n_per_w
        pltpu.sync_copy(
            idx_hbm.at[wid // w_per_row,
                       pl.ds((wid % w_per_row) * n_per_w, n_per_w)],
            idx_v)

        def gstart(c):
            b = c % K
            return pltpu.async_copy(
                table_hbm.at[idx_v.at[pl.ds(c * CH, CH)]],
                rows_v.at[b], gsem.at[b])

        def sstart(c):
            b = c % K
            return pltpu.async_copy(
                rows_v.at[b], out_hbm.at[pl.ds(base + c * CH, CH)],
                ssem.at[b])

        g = {}
        s = {}
        for c in range(min(K - 1, n_ch)):
            g[c] = gstart(c)
        for c in range(n_ch):
            g[c].wait()
            s[c] = sstart(c)
            m = c + K - 1
            if m < n_ch:
                if c >= 1:
                    s[c - 1].wait()
                g[m] = gstart(m)
        for c in range(max(0, n_ch - K), n_ch):
            s[c].wait()

    return emb


def kernel(x, weights):
    B, S = x.shape
    V, D = weights.shape
    info = plsc.get_sparse_core_info()
    emb = _make_emb(B, S, D, info.num_cores, info.num_subcores)
    out = emb(x, weights)
    return out.reshape(B, 1, S, D)
